# baseline (speedup 1.0000x reference)
import functools

import jax
import jax.numpy as jnp
from jax import lax
from jax.experimental import pallas as pl
from jax.experimental.pallas import tpu as pltpu
from jax.experimental.pallas import tpu_sc as plsc

N = 10000
D = 128
E = 320000
N1 = 10112
DUMMY = 10000
NW = 32
NCH = 80
NCH0 = 40
NCH1 = 2 * NCH - NCH0
CH = 128
E_PAD = NW * NCH * CH
E_ROWS = NW * NCH + 1
RPT = N1 // 16
BLK = 400
GRID = N // BLK

_mesh = plsc.VectorSubcoreMesh(core_axis_name="c", subcore_axis_name="s")


def _zero_fill(rows, zvec):
    z16 = jnp.zeros((16,), jnp.float32)
    for g in range(8):
        zvec[pl.ds(16 * g, 16)] = z16

    def zr(i, carry):
        for j in range(8):
            rows[i, pl.ds(16 * j, 16)] = z16
        return carry

    lax.fori_loop(0, CH, zr, 0)


def _zero_spmem(rows, zvec, acc_sh, vec_sh, r0):
    for k in range(4):
        pltpu.sync_copy(rows, acc_sh.at[pl.ds(r0 + 128 * k, 128)])
        pltpu.sync_copy(zvec, vec_sh.at[pl.ds(r0 + 128 * k, 128)])
    pltpu.sync_copy(rows.at[pl.ds(0, 120)], acc_sh.at[pl.ds(r0 + 512, 120)])
    pltpu.sync_copy(zvec.at[pl.ds(0, 120)], vec_sh.at[pl.ds(r0 + 512, 120)])


@functools.partial(
    pl.kernel,
    out_type=(
        jax.ShapeDtypeStruct((2, N1, D), jnp.float32),
        jax.ShapeDtypeStruct((2 * N1,), jnp.float32),
    ),
    mesh=_mesh,
    compiler_params=pltpu.CompilerParams(needs_layout_passes=False),
    scratch_types=(
        pltpu.VMEM((2, CH), jnp.int32),
        pltpu.VMEM((2, CH), jnp.int32),
        pltpu.VMEM((2, CH, D), jnp.float32),
        pltpu.VMEM((CH,), jnp.float32),
        pltpu.VMEM((CH,), jnp.float32),
        pltpu.VMEM((RPT,), jnp.float32),
        pltpu.VMEM_SHARED((N1, D), jnp.float32),
        pltpu.VMEM_SHARED((N1,), jnp.float32),
        pltpu.SemaphoreType.DMA,
        pltpu.SemaphoreType.DMA,
    ),
)
def _sc_sage(xl_hbm, srcc, dstc, acc_out, cnt_out,
             sidx, didx, rows, ones_v, zvec, cbuf, acc_sh, cnt_sh, sem0, sem1):
    c = lax.axis_index("c")
    s = lax.axis_index("s")
    wid = s * 2 + c
    sems = (sem0, sem1)
    one16 = jnp.ones((16,), jnp.float32)
    for g in range(8):
        ones_v[pl.ds(16 * g, 16)] = one16
    _zero_fill(rows.at[0], zvec)
    _zero_fill(rows.at[1], zvec)
    r0 = s * RPT
    _zero_spmem(rows.at[0], zvec, acc_sh, cnt_sh, r0)
    plsc.subcore_barrier()

    base = s * (2 * NCH) + c * NCH0
    nch2 = jnp.where(c == 0, NCH0 // 2, NCH1 // 2)

    def start(j, b):
        row = base + j
        pltpu.sync_copy(srcc.at[row], sidx.at[b])
        pltpu.sync_copy(dstc.at[row], didx.at[b])
        return pltpu.async_copy(xl_hbm.at[sidx.at[b]], rows.at[b], sems[b])

    def drain(b):
        pltpu.make_async_copy(xl_hbm.at[sidx.at[b]], rows.at[b], sems[b]).wait()

    def consume(b):
        drain(b)
        pltpu.sync_copy(rows.at[b], acc_sh.at[didx.at[b]], add=True)
        pltpu.sync_copy(ones_v, cnt_sh.at[didx.at[b]], add=True)

    start(0, 0)

    def body(k, carry):
        start(2 * k + 1, 1)
        consume(0)
        start(2 * k + 2, 0)
        consume(1)
        return carry

    lax.fori_loop(0, nch2, body, 0)
    drain(0)
    plsc.subcore_barrier()
    pltpu.sync_copy(acc_sh.at[pl.ds(r0, RPT)], acc_out.at[c, pl.ds(r0, RPT)])
    pltpu.sync_copy(cnt_sh.at[pl.ds(r0, RPT)], cbuf)
    pltpu.sync_copy(cbuf, cnt_out.at[pl.ds(c * N1 + r0, RPT)])


@functools.partial(
    pl.kernel,
    out_type=(
        jax.ShapeDtypeStruct((2, N1, D), jnp.float32),
        jax.ShapeDtypeStruct((2 * N1,), jnp.float32),
    ),
    mesh=_mesh,
    compiler_params=pltpu.CompilerParams(needs_layout_passes=False),
    scratch_types=(
        pltpu.VMEM((2, CH), jnp.int32),
        pltpu.VMEM((2, CH), jnp.int32),
        pltpu.VMEM((2, CH, D), jnp.float32),
        pltpu.VMEM((2, CH), jnp.float32),
        pltpu.VMEM((2, CH), jnp.float32),
        pltpu.VMEM((CH,), jnp.float32),
        pltpu.VMEM((N1,), jnp.float32),
        pltpu.VMEM((16,), jnp.float32),
        pltpu.VMEM((RPT,), jnp.float32),
        pltpu.VMEM_SHARED((N1, D), jnp.float32),
        pltpu.VMEM_SHARED((N1,), jnp.float32),
        pltpu.SemaphoreType.DMA,
        pltpu.SemaphoreType.DMA,
        pltpu.SemaphoreType.DMA,
        pltpu.SemaphoreType.DMA,
    ),
)
def _sc_gat(hg_hbm, srcc, dstc, as_hbm, ad_hbm, ma_hbm, acc_out, den_out,
            sidx, didx, rows, wbuf, asb, zvec, adv, mxv, cbuf, acc_sh, den_sh,
            sem0, sem1, semA0, semA1):
    c = lax.axis_index("c")
    s = lax.axis_index("s")
    wid = s * 2 + c
    sems = (sem0, sem1)
    asems = (semA0, semA1)
    pltpu.sync_copy(ad_hbm, adv)
    pltpu.sync_copy(ma_hbm, mxv)
    _zero_fill(rows.at[0], zvec)
    _zero_fill(rows.at[1], zvec)
    r0 = s * RPT
    _zero_spmem(rows.at[0], zvec, acc_sh, den_sh, r0)
    plsc.subcore_barrier()

    base = s * (2 * NCH) + c * NCH0
    nch2 = jnp.where(c == 0, NCH0 // 2, NCH1 // 2)

    def start(j, b):
        row = base + j
        pltpu.sync_copy(srcc.at[row], sidx.at[b])
        pltpu.sync_copy(dstc.at[row], didx.at[b])
        pltpu.async_copy(as_hbm.at[sidx.at[b]], asb.at[b], asems[b])
        return pltpu.async_copy(hg_hbm.at[sidx.at[b]], rows.at[b], sems[b])

    def drain(b):
        pltpu.make_async_copy(as_hbm.at[sidx.at[b]], asb.at[b], asems[b]).wait()
        pltpu.make_async_copy(hg_hbm.at[sidx.at[b]], rows.at[b], sems[b]).wait()

    def consume(b):
        pltpu.make_async_copy(as_hbm.at[sidx.at[b]], asb.at[b], asems[b]).wait()
        mx16 = mxv[pl.ds(0, 16)]
        for g in range(8):
            d16 = didx[b, pl.ds(16 * g, 16)]
            av = asb[b, pl.ds(16 * g, 16)]
            dv = plsc.load_gather(adv, [d16])
            t = av + dv
            e = jnp.where(t > 0, t, 0.2 * t)
            t2 = mx16 + dv
            mg = jnp.where(t2 > 0, t2, 0.2 * t2)
            wbuf[b, pl.ds(16 * g, 16)] = jnp.exp(e - mg)
        pltpu.make_async_copy(hg_hbm.at[sidx.at[b]], rows.at[b], sems[b]).wait()
        for g in range(8):
            w16 = wbuf[b, pl.ds(16 * g, 16)]
            for e in range(16):
                wv = w16[e]
                i = 16 * g + e
                for jj in range(8):
                    rows[b, i, pl.ds(16 * jj, 16)] = rows[b, i, pl.ds(16 * jj, 16)] * wv
        pltpu.sync_copy(rows.at[b], acc_sh.at[didx.at[b]], add=True)
        pltpu.sync_copy(wbuf.at[b], den_sh.at[didx.at[b]], add=True)

    start(0, 0)

    def body(k, carry):
        start(2 * k + 1, 1)
        consume(0)
        start(2 * k + 2, 0)
        consume(1)
        return carry

    lax.fori_loop(0, nch2, body, 0)
    drain(0)
    plsc.subcore_barrier()
    pltpu.sync_copy(acc_sh.at[pl.ds(r0, RPT)], acc_out.at[c, pl.ds(r0, RPT)])
    pltpu.sync_copy(den_sh.at[pl.ds(r0, RPT)], cbuf)
    pltpu.sync_copy(cbuf, den_out.at[pl.ds(c * N1 + r0, RPT)])


def _k1_body(x_ref, wl_ref, wr_ref, xl_ref, xr_ref):
    xb = x_ref[...]
    dn = (((1,), (1,)), ((), ()))
    xl_ref[...] = lax.dot_general(xb, wl_ref[...], dn, preferred_element_type=jnp.float32)
    xr_ref[...] = lax.dot_general(xb, wr_ref[...], dn, preferred_element_type=jnp.float32)


def _k1(x, wl, wr):
    return pl.pallas_call(
        _k1_body,
        out_shape=(
            jax.ShapeDtypeStruct((N, D), jnp.float32),
            jax.ShapeDtypeStruct((N, D), jnp.float32),
        ),
        grid=(GRID,),
        in_specs=[
            pl.BlockSpec((BLK, D), lambda i: (i, 0)),
            pl.BlockSpec((D, D), lambda i: (0, 0)),
            pl.BlockSpec((D, D), lambda i: (0, 0)),
        ],
        out_specs=(
            pl.BlockSpec((BLK, D), lambda i: (i, 0)),
            pl.BlockSpec((BLK, D), lambda i: (i, 0)),
        ),
    )(x, wl, wr)


def _k2_body(p_ref, cnt_ref, xr_ref, bl_ref, gw_ref, asrc_ref, adst_ref,
             hg_ref, as_ref, ad_ref):
    p = p_ref[0] + p_ref[1]
    cnt = cnt_ref[0, :, 0] + cnt_ref[1, :, 0]
    mean = p / jnp.maximum(cnt, 1.0)[:, None]
    h = jnp.maximum(mean + bl_ref[...][None, :] + xr_ref[...], 0.0)
    dn = (((1,), (1,)), ((), ()))
    hg = lax.dot_general(h, gw_ref[...], dn, preferred_element_type=jnp.float32)
    hg_ref[...] = hg
    dv = (((1,), (0,)), ((), ()))
    as_ref[...] = lax.dot_general(hg, asrc_ref[...], dv, preferred_element_type=jnp.float32)[:, None]
    ad_ref[...] = lax.dot_general(hg, adst_ref[...], dv, preferred_element_type=jnp.float32)[:, None]


def _k2(acc_p, cnt_p, xr, bl, gw, asrc, adst):
    return pl.pallas_call(
        _k2_body,
        out_shape=(
            jax.ShapeDtypeStruct((N, D), jnp.float32),
            jax.ShapeDtypeStruct((N, 1), jnp.float32),
            jax.ShapeDtypeStruct((N, 1), jnp.float32),
        ),
        grid=(GRID,),
        in_specs=[
            pl.BlockSpec((2, BLK, D), lambda i: (0, i, 0)),
            pl.BlockSpec((2, BLK, 1), lambda i: (0, i, 0)),
            pl.BlockSpec((BLK, D), lambda i: (i, 0)),
            pl.BlockSpec((D,), lambda i: (0,)),
            pl.BlockSpec((D, D), lambda i: (0, 0)),
            pl.BlockSpec((D,), lambda i: (0,)),
            pl.BlockSpec((D,), lambda i: (0,)),
        ],
        out_specs=(
            pl.BlockSpec((BLK, D), lambda i: (i, 0)),
            pl.BlockSpec((BLK, 1), lambda i: (i, 0)),
            pl.BlockSpec((BLK, 1), lambda i: (i, 0)),
        ),
    )(acc_p, cnt_p, xr, bl, gw, asrc, adst)


def _k2b_body(as_ref, ad_ref, wl_ref, ma_ref):
    a_s = as_ref[...]
    a_d = ad_ref[...]
    max_a = jnp.max(a_s)
    t = a_s + a_d
    el = jnp.where(t > 0, t, 0.2 * t)
    t2 = max_a + a_d
    m = jnp.where(t2 > 0, t2, 0.2 * t2)
    wl_ref[...] = jnp.exp(el - m)
    ma_ref[...] = jnp.full((1, 16), max_a, jnp.float32)


def _k2b(a_s, a_d):
    return pl.pallas_call(
        _k2b_body,
        out_shape=(
            jax.ShapeDtypeStruct((N, 1), jnp.float32),
            jax.ShapeDtypeStruct((1, 16), jnp.float32),
        ),
    )(a_s, a_d)


def _k3_body(s_ref, den_ref, wl_ref, hg_ref, bias_ref, o_ref):
    wl = wl_ref[...]
    num = s_ref[0] + s_ref[1] + wl * hg_ref[...]
    den = den_ref[0, :, 0] + den_ref[1, :, 0] + wl[:, 0] + 1e-16
    out = num / den[:, None] + bias_ref[...][None, :]
    mx = jnp.max(out, axis=1, keepdims=True)
    o_ref[...] = out - mx - jnp.log(jnp.sum(jnp.exp(out - mx), axis=1, keepdims=True))


def _k3(gacc_p, den_p, w_loop, hg, bias):
    return pl.pallas_call(
        _k3_body,
        out_shape=jax.ShapeDtypeStruct((N, D), jnp.float32),
        grid=(GRID,),
        in_specs=[
            pl.BlockSpec((2, BLK, D), lambda i: (0, i, 0)),
            pl.BlockSpec((2, BLK, 1), lambda i: (0, i, 0)),
            pl.BlockSpec((BLK, 1), lambda i: (i, 0)),
            pl.BlockSpec((BLK, D), lambda i: (i, 0)),
            pl.BlockSpec((D,), lambda i: (0,)),
        ],
        out_specs=pl.BlockSpec((BLK, D), lambda i: (i, 0)),
    )(gacc_p, den_p, w_loop, hg, bias)


def kernel(x, edge_index, sage_w_l, sage_b_l, sage_w_r, gat_w, gat_att_src, gat_att_dst, gat_bias):
    src = edge_index[0].astype(jnp.int32)
    dst = edge_index[1].astype(jnp.int32)
    pad = jnp.full((E_ROWS * CH - E,), DUMMY, jnp.int32)
    srcc = jnp.concatenate([src, pad]).reshape(E_ROWS, CH)
    dstc = jnp.concatenate([dst, pad]).reshape(E_ROWS, CH)

    xl, xr = _k1(x, sage_w_l, sage_w_r)
    xl_pad = jnp.pad(xl, ((0, N1 - N), (0, 0)))
    acc_p, cnt_p = _sc_sage(xl_pad, srcc, dstc)

    hg, a_s, a_d = _k2(acc_p, cnt_p.reshape(2, N1, 1), xr, sage_b_l,
                       gat_w, gat_att_src, gat_att_dst)
    w_loop, maxa = _k2b(a_s, a_d)

    hg_pad = jnp.pad(hg, ((0, N1 - N), (0, 0)))
    as_pad = jnp.pad(a_s[:, 0], (0, N1 - N))
    ad_pad = jnp.pad(a_d[:, 0], (0, N1 - N))
    gacc_p, den_p = _sc_gat(hg_pad, srcc, dstc, as_pad, ad_pad, maxa[0])

    return _k3(gacc_p, den_p.reshape(2, N1, 1), w_loop, hg, gat_bias)


# speedup vs baseline: 1.2418x; 1.2418x over previous
import functools

import jax
import jax.numpy as jnp
from jax import lax
from jax.experimental import pallas as pl
from jax.experimental.pallas import tpu as pltpu
from jax.experimental.pallas import tpu_sc as plsc

N = 10000
D = 128
E = 320000
N1 = 10112
DUMMY = 10000
NW = 32
NCH = 80
NCH0 = 120
NCH1 = 2 * NCH - NCH0
CH = 128
E_PAD = NW * NCH * CH
E_ROWS = NW * NCH + 1
RPT = N1 // 16
BLK = 400
GRID = N // BLK

_mesh = plsc.VectorSubcoreMesh(core_axis_name="c", subcore_axis_name="s")


def _zero_fill(rows, zvec):
    z16 = jnp.zeros((16,), jnp.float32)
    for g in range(8):
        zvec[pl.ds(16 * g, 16)] = z16

    def zr(i, carry):
        for j in range(8):
            rows[i, pl.ds(16 * j, 16)] = z16
        return carry

    lax.fori_loop(0, CH, zr, 0)


def _zero_spmem(rows, zvec, acc_sh, vec_sh, r0):
    for k in range(4):
        pltpu.sync_copy(rows, acc_sh.at[pl.ds(r0 + 128 * k, 128)])
        pltpu.sync_copy(zvec, vec_sh.at[pl.ds(r0 + 128 * k, 128)])
    pltpu.sync_copy(rows.at[pl.ds(0, 120)], acc_sh.at[pl.ds(r0 + 512, 120)])
    pltpu.sync_copy(zvec.at[pl.ds(0, 120)], vec_sh.at[pl.ds(r0 + 512, 120)])


@functools.partial(
    pl.kernel,
    out_type=(
        jax.ShapeDtypeStruct((2, N1, D), jnp.float32),
        jax.ShapeDtypeStruct((2 * N1,), jnp.float32),
    ),
    mesh=_mesh,
    compiler_params=pltpu.CompilerParams(needs_layout_passes=False),
    scratch_types=(
        pltpu.VMEM((2, CH), jnp.int32),
        pltpu.VMEM((2, CH), jnp.int32),
        pltpu.VMEM((2, CH, D), jnp.float32),
        pltpu.VMEM((CH,), jnp.float32),
        pltpu.VMEM((CH,), jnp.float32),
        pltpu.VMEM((RPT,), jnp.float32),
        pltpu.VMEM_SHARED((N1, D), jnp.float32),
        pltpu.VMEM_SHARED((N1,), jnp.float32),
        pltpu.SemaphoreType.DMA,
        pltpu.SemaphoreType.DMA,
    ),
)
def _sc_sage(xl_hbm, srcc, dstc, acc_out, cnt_out,
             sidx, didx, rows, ones_v, zvec, cbuf, acc_sh, cnt_sh, sem0, sem1):
    c = lax.axis_index("c")
    s = lax.axis_index("s")
    wid = s * 2 + c
    sems = (sem0, sem1)
    one16 = jnp.ones((16,), jnp.float32)
    for g in range(8):
        ones_v[pl.ds(16 * g, 16)] = one16
    _zero_fill(rows.at[0], zvec)
    _zero_fill(rows.at[1], zvec)
    r0 = s * RPT
    _zero_spmem(rows.at[0], zvec, acc_sh, cnt_sh, r0)
    plsc.subcore_barrier()

    base = s * (2 * NCH) + c * NCH0
    nch2 = jnp.where(c == 0, NCH0 // 2, NCH1 // 2)

    def start(j, b):
        row = base + j
        pltpu.sync_copy(srcc.at[row], sidx.at[b])
        pltpu.sync_copy(dstc.at[row], didx.at[b])
        return pltpu.async_copy(xl_hbm.at[sidx.at[b]], rows.at[b], sems[b])

    def drain(b):
        pltpu.make_async_copy(xl_hbm.at[sidx.at[b]], rows.at[b], sems[b]).wait()

    def consume(b):
        drain(b)
        pltpu.sync_copy(rows.at[b], acc_sh.at[didx.at[b]], add=True)
        pltpu.sync_copy(ones_v, cnt_sh.at[didx.at[b]], add=True)

    start(0, 0)

    def body(k, carry):
        start(2 * k + 1, 1)
        consume(0)
        start(2 * k + 2, 0)
        consume(1)
        return carry

    lax.fori_loop(0, nch2, body, 0)
    drain(0)
    plsc.subcore_barrier()
    pltpu.sync_copy(acc_sh.at[pl.ds(r0, RPT)], acc_out.at[c, pl.ds(r0, RPT)])
    pltpu.sync_copy(cnt_sh.at[pl.ds(r0, RPT)], cbuf)
    pltpu.sync_copy(cbuf, cnt_out.at[pl.ds(c * N1 + r0, RPT)])


@functools.partial(
    pl.kernel,
    out_type=(
        jax.ShapeDtypeStruct((2, N1, D), jnp.float32),
        jax.ShapeDtypeStruct((2 * N1,), jnp.float32),
    ),
    mesh=_mesh,
    compiler_params=pltpu.CompilerParams(needs_layout_passes=False),
    scratch_types=(
        pltpu.VMEM((2, CH), jnp.int32),
        pltpu.VMEM((2, CH), jnp.int32),
        pltpu.VMEM((2, CH, D), jnp.float32),
        pltpu.VMEM((2, CH), jnp.float32),
        pltpu.VMEM((2, CH), jnp.float32),
        pltpu.VMEM((CH,), jnp.float32),
        pltpu.VMEM((N1,), jnp.float32),
        pltpu.VMEM((16,), jnp.float32),
        pltpu.VMEM((RPT,), jnp.float32),
        pltpu.VMEM_SHARED((N1, D), jnp.float32),
        pltpu.VMEM_SHARED((N1,), jnp.float32),
        pltpu.SemaphoreType.DMA,
        pltpu.SemaphoreType.DMA,
        pltpu.SemaphoreType.DMA,
        pltpu.SemaphoreType.DMA,
    ),
)
def _sc_gat(hg_hbm, srcc, dstc, as_hbm, ad_hbm, ma_hbm, acc_out, den_out,
            sidx, didx, rows, wbuf, asb, zvec, adv, mxv, cbuf, acc_sh, den_sh,
            sem0, sem1, semA0, semA1):
    c = lax.axis_index("c")
    s = lax.axis_index("s")
    wid = s * 2 + c
    sems = (sem0, sem1)
    asems = (semA0, semA1)
    pltpu.sync_copy(ad_hbm, adv)
    pltpu.sync_copy(ma_hbm, mxv)
    _zero_fill(rows.at[0], zvec)
    _zero_fill(rows.at[1], zvec)
    r0 = s * RPT
    _zero_spmem(rows.at[0], zvec, acc_sh, den_sh, r0)
    plsc.subcore_barrier()

    base = s * (2 * NCH) + c * NCH0
    nch2 = jnp.where(c == 0, NCH0 // 2, NCH1 // 2)

    def start(j, b):
        row = base + j
        pltpu.sync_copy(srcc.at[row], sidx.at[b])
        pltpu.sync_copy(dstc.at[row], didx.at[b])
        pltpu.async_copy(as_hbm.at[sidx.at[b]], asb.at[b], asems[b])
        return pltpu.async_copy(hg_hbm.at[sidx.at[b]], rows.at[b], sems[b])

    def drain(b):
        pltpu.make_async_copy(as_hbm.at[sidx.at[b]], asb.at[b], asems[b]).wait()
        pltpu.make_async_copy(hg_hbm.at[sidx.at[b]], rows.at[b], sems[b]).wait()

    def consume(b):
        pltpu.make_async_copy(as_hbm.at[sidx.at[b]], asb.at[b], asems[b]).wait()
        mx16 = mxv[pl.ds(0, 16)]
        for g in range(8):
            d16 = didx[b, pl.ds(16 * g, 16)]
            av = asb[b, pl.ds(16 * g, 16)]
            dv = plsc.load_gather(adv, [d16])
            t = av + dv
            e = jnp.where(t > 0, t, 0.2 * t)
            t2 = mx16 + dv
            mg = jnp.where(t2 > 0, t2, 0.2 * t2)
            wbuf[b, pl.ds(16 * g, 16)] = jnp.exp(e - mg)
        pltpu.make_async_copy(hg_hbm.at[sidx.at[b]], rows.at[b], sems[b]).wait()
        for g in range(8):
            w16 = wbuf[b, pl.ds(16 * g, 16)]
            for e in range(16):
                wv = w16[e]
                i = 16 * g + e
                for jj in range(8):
                    rows[b, i, pl.ds(16 * jj, 16)] = rows[b, i, pl.ds(16 * jj, 16)] * wv
        pltpu.sync_copy(rows.at[b], acc_sh.at[didx.at[b]], add=True)
        pltpu.sync_copy(wbuf.at[b], den_sh.at[didx.at[b]], add=True)

    start(0, 0)

    def body(k, carry):
        start(2 * k + 1, 1)
        consume(0)
        start(2 * k + 2, 0)
        consume(1)
        return carry

    lax.fori_loop(0, nch2, body, 0)
    drain(0)
    plsc.subcore_barrier()
    pltpu.sync_copy(acc_sh.at[pl.ds(r0, RPT)], acc_out.at[c, pl.ds(r0, RPT)])
    pltpu.sync_copy(den_sh.at[pl.ds(r0, RPT)], cbuf)
    pltpu.sync_copy(cbuf, den_out.at[pl.ds(c * N1 + r0, RPT)])


def _k1_body(x_ref, wl_ref, wr_ref, xl_ref, xr_ref):
    xb = x_ref[...]
    dn = (((1,), (1,)), ((), ()))
    xl_ref[...] = lax.dot_general(xb, wl_ref[...], dn, preferred_element_type=jnp.float32)
    xr_ref[...] = lax.dot_general(xb, wr_ref[...], dn, preferred_element_type=jnp.float32)


def _k1(x, wl, wr):
    return pl.pallas_call(
        _k1_body,
        out_shape=(
            jax.ShapeDtypeStruct((N, D), jnp.float32),
            jax.ShapeDtypeStruct((N, D), jnp.float32),
        ),
        grid=(GRID,),
        in_specs=[
            pl.BlockSpec((BLK, D), lambda i: (i, 0)),
            pl.BlockSpec((D, D), lambda i: (0, 0)),
            pl.BlockSpec((D, D), lambda i: (0, 0)),
        ],
        out_specs=(
            pl.BlockSpec((BLK, D), lambda i: (i, 0)),
            pl.BlockSpec((BLK, D), lambda i: (i, 0)),
        ),
    )(x, wl, wr)


def _k2_body(p_ref, cnt_ref, xr_ref, bl_ref, gw_ref, asrc_ref, adst_ref,
             hg_ref, as_ref, ad_ref):
    p = p_ref[0] + p_ref[1]
    cnt = cnt_ref[0, :, 0] + cnt_ref[1, :, 0]
    mean = p / jnp.maximum(cnt, 1.0)[:, None]
    h = jnp.maximum(mean + bl_ref[...][None, :] + xr_ref[...], 0.0)
    dn = (((1,), (1,)), ((), ()))
    hg = lax.dot_general(h, gw_ref[...], dn, preferred_element_type=jnp.float32)
    hg_ref[...] = hg
    dv = (((1,), (0,)), ((), ()))
    as_ref[...] = lax.dot_general(hg, asrc_ref[...], dv, preferred_element_type=jnp.float32)[:, None]
    ad_ref[...] = lax.dot_general(hg, adst_ref[...], dv, preferred_element_type=jnp.float32)[:, None]


def _k2(acc_p, cnt_p, xr, bl, gw, asrc, adst):
    return pl.pallas_call(
        _k2_body,
        out_shape=(
            jax.ShapeDtypeStruct((N, D), jnp.float32),
            jax.ShapeDtypeStruct((N, 1), jnp.float32),
            jax.ShapeDtypeStruct((N, 1), jnp.float32),
        ),
        grid=(GRID,),
        in_specs=[
            pl.BlockSpec((2, BLK, D), lambda i: (0, i, 0)),
            pl.BlockSpec((2, BLK, 1), lambda i: (0, i, 0)),
            pl.BlockSpec((BLK, D), lambda i: (i, 0)),
            pl.BlockSpec((D,), lambda i: (0,)),
            pl.BlockSpec((D, D), lambda i: (0, 0)),
            pl.BlockSpec((D,), lambda i: (0,)),
            pl.BlockSpec((D,), lambda i: (0,)),
        ],
        out_specs=(
            pl.BlockSpec((BLK, D), lambda i: (i, 0)),
            pl.BlockSpec((BLK, 1), lambda i: (i, 0)),
            pl.BlockSpec((BLK, 1), lambda i: (i, 0)),
        ),
    )(acc_p, cnt_p, xr, bl, gw, asrc, adst)


def _k2b_body(as_ref, ad_ref, wl_ref, ma_ref):
    a_s = as_ref[...]
    a_d = ad_ref[...]
    max_a = jnp.max(a_s)
    t = a_s + a_d
    el = jnp.where(t > 0, t, 0.2 * t)
    t2 = max_a + a_d
    m = jnp.where(t2 > 0, t2, 0.2 * t2)
    wl_ref[...] = jnp.exp(el - m)
    ma_ref[...] = jnp.full((1, 16), max_a, jnp.float32)


def _k2b(a_s, a_d):
    return pl.pallas_call(
        _k2b_body,
        out_shape=(
            jax.ShapeDtypeStruct((N, 1), jnp.float32),
            jax.ShapeDtypeStruct((1, 16), jnp.float32),
        ),
    )(a_s, a_d)


def _k3_body(s_ref, den_ref, wl_ref, hg_ref, bias_ref, o_ref):
    wl = wl_ref[...]
    num = s_ref[0] + s_ref[1] + wl * hg_ref[...]
    den = den_ref[0, :, 0] + den_ref[1, :, 0] + wl[:, 0] + 1e-16
    out = num / den[:, None] + bias_ref[...][None, :]
    mx = jnp.max(out, axis=1, keepdims=True)
    o_ref[...] = out - mx - jnp.log(jnp.sum(jnp.exp(out - mx), axis=1, keepdims=True))


def _k3(gacc_p, den_p, w_loop, hg, bias):
    return pl.pallas_call(
        _k3_body,
        out_shape=jax.ShapeDtypeStruct((N, D), jnp.float32),
        grid=(GRID,),
        in_specs=[
            pl.BlockSpec((2, BLK, D), lambda i: (0, i, 0)),
            pl.BlockSpec((2, BLK, 1), lambda i: (0, i, 0)),
            pl.BlockSpec((BLK, 1), lambda i: (i, 0)),
            pl.BlockSpec((BLK, D), lambda i: (i, 0)),
            pl.BlockSpec((D,), lambda i: (0,)),
        ],
        out_specs=pl.BlockSpec((BLK, D), lambda i: (i, 0)),
    )(gacc_p, den_p, w_loop, hg, bias)


def kernel(x, edge_index, sage_w_l, sage_b_l, sage_w_r, gat_w, gat_att_src, gat_att_dst, gat_bias):
    src = edge_index[0].astype(jnp.int32)
    dst = edge_index[1].astype(jnp.int32)
    pad = jnp.full((E_ROWS * CH - E,), DUMMY, jnp.int32)
    srcc = jnp.concatenate([src, pad]).reshape(E_ROWS, CH)
    dstc = jnp.concatenate([dst, pad]).reshape(E_ROWS, CH)

    xl, xr = _k1(x, sage_w_l, sage_w_r)
    xl_pad = jnp.pad(xl, ((0, N1 - N), (0, 0)))
    acc_p, cnt_p = _sc_sage(xl_pad, srcc, dstc)

    hg, a_s, a_d = _k2(acc_p, cnt_p.reshape(2, N1, 1), xr, sage_b_l,
                       gat_w, gat_att_src, gat_att_dst)
    w_loop, maxa = _k2b(a_s, a_d)

    hg_pad = jnp.pad(hg, ((0, N1 - N), (0, 0)))
    as_pad = jnp.pad(a_s[:, 0], (0, N1 - N))
    ad_pad = jnp.pad(a_d[:, 0], (0, N1 - N))
    gacc_p, den_p = _sc_gat(hg_pad, srcc, dstc, as_pad, ad_pad, maxa[0])

    return _k3(gacc_p, den_p.reshape(2, N1, 1), w_loop, hg, gat_bias)


# speedup vs baseline: 1.3104x; 1.0552x over previous
import functools

import jax
import jax.numpy as jnp
from jax import lax
from jax.experimental import pallas as pl
from jax.experimental.pallas import tpu as pltpu
from jax.experimental.pallas import tpu_sc as plsc

N = 10000
D = 128
E = 320000
N1 = 10112
DUMMY = 10000
NW = 32
NCH = 80
NCH0_1 = 120
NCH1_1 = 2 * NCH - NCH0_1
NCH0_2 = 94
NCH1_2 = 2 * NCH - NCH0_2
CH = 128
E_PAD = NW * NCH * CH
E_ROWS = NW * NCH + 1
RPT = N1 // 16
BLK = 400
GRID = N // BLK

_mesh = plsc.VectorSubcoreMesh(core_axis_name="c", subcore_axis_name="s")


def _zero_fill(rows, zvec):
    z16 = jnp.zeros((16,), jnp.float32)
    for g in range(8):
        zvec[pl.ds(16 * g, 16)] = z16

    def zr(i, carry):
        for j in range(8):
            rows[i, pl.ds(16 * j, 16)] = z16
        return carry

    lax.fori_loop(0, CH, zr, 0)


def _zero_spmem(rows, zvec, acc_sh, vec_sh, r0):
    for k in range(4):
        pltpu.sync_copy(rows, acc_sh.at[pl.ds(r0 + 128 * k, 128)])
        pltpu.sync_copy(zvec, vec_sh.at[pl.ds(r0 + 128 * k, 128)])
    pltpu.sync_copy(rows.at[pl.ds(0, 120)], acc_sh.at[pl.ds(r0 + 512, 120)])
    pltpu.sync_copy(zvec.at[pl.ds(0, 120)], vec_sh.at[pl.ds(r0 + 512, 120)])


@functools.partial(
    pl.kernel,
    out_type=(
        jax.ShapeDtypeStruct((2, N1, D), jnp.float32),
        jax.ShapeDtypeStruct((2 * N1,), jnp.float32),
    ),
    mesh=_mesh,
    compiler_params=pltpu.CompilerParams(needs_layout_passes=False),
    scratch_types=(
        pltpu.VMEM((2, CH), jnp.int32),
        pltpu.VMEM((2, CH), jnp.int32),
        pltpu.VMEM((2, CH, D), jnp.float32),
        pltpu.VMEM((CH,), jnp.float32),
        pltpu.VMEM((CH,), jnp.float32),
        pltpu.VMEM((RPT,), jnp.float32),
        pltpu.VMEM_SHARED((N1, D), jnp.float32),
        pltpu.VMEM_SHARED((N1,), jnp.float32),
        pltpu.SemaphoreType.DMA,
        pltpu.SemaphoreType.DMA,
    ),
)
def _sc_sage(xl_hbm, srcc, dstc, acc_out, cnt_out,
             sidx, didx, rows, ones_v, zvec, cbuf, acc_sh, cnt_sh, sem0, sem1):
    c = lax.axis_index("c")
    s = lax.axis_index("s")
    wid = s * 2 + c
    sems = (sem0, sem1)
    one16 = jnp.ones((16,), jnp.float32)
    for g in range(8):
        ones_v[pl.ds(16 * g, 16)] = one16
    _zero_fill(rows.at[0], zvec)
    _zero_fill(rows.at[1], zvec)
    r0 = s * RPT
    _zero_spmem(rows.at[0], zvec, acc_sh, cnt_sh, r0)
    plsc.subcore_barrier()

    base = s * (2 * NCH) + c * NCH0_1
    nch2 = jnp.where(c == 0, NCH0_1 // 2, NCH1_1 // 2)

    def start(j, b):
        row = base + j
        pltpu.sync_copy(srcc.at[row], sidx.at[b])
        pltpu.sync_copy(dstc.at[row], didx.at[b])
        return pltpu.async_copy(xl_hbm.at[sidx.at[b]], rows.at[b], sems[b])

    def drain(b):
        pltpu.make_async_copy(xl_hbm.at[sidx.at[b]], rows.at[b], sems[b]).wait()

    def consume(b):
        drain(b)
        pltpu.sync_copy(rows.at[b], acc_sh.at[didx.at[b]], add=True)
        pltpu.sync_copy(ones_v, cnt_sh.at[didx.at[b]], add=True)

    start(0, 0)

    def body(k, carry):
        start(2 * k + 1, 1)
        consume(0)
        start(2 * k + 2, 0)
        consume(1)
        return carry

    lax.fori_loop(0, nch2, body, 0)
    drain(0)
    plsc.subcore_barrier()
    pltpu.sync_copy(acc_sh.at[pl.ds(r0, RPT)], acc_out.at[c, pl.ds(r0, RPT)])
    pltpu.sync_copy(cnt_sh.at[pl.ds(r0, RPT)], cbuf)
    pltpu.sync_copy(cbuf, cnt_out.at[pl.ds(c * N1 + r0, RPT)])


@functools.partial(
    pl.kernel,
    out_type=(
        jax.ShapeDtypeStruct((2, N1, D), jnp.float32),
        jax.ShapeDtypeStruct((2 * N1,), jnp.float32),
    ),
    mesh=_mesh,
    compiler_params=pltpu.CompilerParams(needs_layout_passes=False),
    scratch_types=(
        pltpu.VMEM((2, CH), jnp.int32),
        pltpu.VMEM((2, CH), jnp.int32),
        pltpu.VMEM((2, CH, D), jnp.float32),
        pltpu.VMEM((2, CH), jnp.float32),
        pltpu.VMEM((2, CH), jnp.float32),
        pltpu.VMEM((CH,), jnp.float32),
        pltpu.VMEM((N1,), jnp.float32),
        pltpu.VMEM((16,), jnp.float32),
        pltpu.VMEM((RPT,), jnp.float32),
        pltpu.VMEM_SHARED((N1, D), jnp.float32),
        pltpu.VMEM_SHARED((N1,), jnp.float32),
        pltpu.SemaphoreType.DMA,
        pltpu.SemaphoreType.DMA,
        pltpu.SemaphoreType.DMA,
        pltpu.SemaphoreType.DMA,
    ),
)
def _sc_gat(hg_hbm, srcc, dstc, as_hbm, ad_hbm, ma_hbm, acc_out, den_out,
            sidx, didx, rows, wbuf, asb, zvec, adv, mxv, cbuf, acc_sh, den_sh,
            sem0, sem1, semA0, semA1):
    c = lax.axis_index("c")
    s = lax.axis_index("s")
    wid = s * 2 + c
    sems = (sem0, sem1)
    asems = (semA0, semA1)
    pltpu.sync_copy(ad_hbm, adv)
    pltpu.sync_copy(ma_hbm, mxv)
    _zero_fill(rows.at[0], zvec)
    _zero_fill(rows.at[1], zvec)
    r0 = s * RPT
    _zero_spmem(rows.at[0], zvec, acc_sh, den_sh, r0)
    plsc.subcore_barrier()

    base = s * (2 * NCH) + c * NCH0_2
    nch2 = jnp.where(c == 0, NCH0_2 // 2, NCH1_2 // 2)

    def start(j, b):
        row = base + j
        pltpu.sync_copy(srcc.at[row], sidx.at[b])
        pltpu.sync_copy(dstc.at[row], didx.at[b])
        pltpu.async_copy(as_hbm.at[sidx.at[b]], asb.at[b], asems[b])
        return pltpu.async_copy(hg_hbm.at[sidx.at[b]], rows.at[b], sems[b])

    def drain(b):
        pltpu.make_async_copy(as_hbm.at[sidx.at[b]], asb.at[b], asems[b]).wait()
        pltpu.make_async_copy(hg_hbm.at[sidx.at[b]], rows.at[b], sems[b]).wait()

    def consume(b):
        pltpu.make_async_copy(as_hbm.at[sidx.at[b]], asb.at[b], asems[b]).wait()
        mx16 = mxv[pl.ds(0, 16)]
        for g in range(8):
            d16 = didx[b, pl.ds(16 * g, 16)]
            av = asb[b, pl.ds(16 * g, 16)]
            dv = plsc.load_gather(adv, [d16])
            t = av + dv
            e = jnp.where(t > 0, t, 0.2 * t)
            t2 = mx16 + dv
            mg = jnp.where(t2 > 0, t2, 0.2 * t2)
            wbuf[b, pl.ds(16 * g, 16)] = jnp.exp(e - mg)
        pltpu.make_async_copy(hg_hbm.at[sidx.at[b]], rows.at[b], sems[b]).wait()
        for g in range(8):
            w16 = wbuf[b, pl.ds(16 * g, 16)]
            for e in range(16):
                wv = w16[e]
                i = 16 * g + e
                for jj in range(8):
                    rows[b, i, pl.ds(16 * jj, 16)] = rows[b, i, pl.ds(16 * jj, 16)] * wv
        pltpu.sync_copy(rows.at[b], acc_sh.at[didx.at[b]], add=True)
        pltpu.sync_copy(wbuf.at[b], den_sh.at[didx.at[b]], add=True)

    start(0, 0)

    def body(k, carry):
        start(2 * k + 1, 1)
        consume(0)
        start(2 * k + 2, 0)
        consume(1)
        return carry

    lax.fori_loop(0, nch2, body, 0)
    drain(0)
    plsc.subcore_barrier()
    pltpu.sync_copy(acc_sh.at[pl.ds(r0, RPT)], acc_out.at[c, pl.ds(r0, RPT)])
    pltpu.sync_copy(den_sh.at[pl.ds(r0, RPT)], cbuf)
    pltpu.sync_copy(cbuf, den_out.at[pl.ds(c * N1 + r0, RPT)])


def _k1_body(x_ref, wl_ref, wr_ref, xl_ref, xr_ref):
    xb = x_ref[...]
    dn = (((1,), (1,)), ((), ()))
    xl_ref[...] = lax.dot_general(xb, wl_ref[...], dn, preferred_element_type=jnp.float32)
    xr_ref[...] = lax.dot_general(xb, wr_ref[...], dn, preferred_element_type=jnp.float32)


def _k1(x, wl, wr):
    return pl.pallas_call(
        _k1_body,
        out_shape=(
            jax.ShapeDtypeStruct((N, D), jnp.float32),
            jax.ShapeDtypeStruct((N, D), jnp.float32),
        ),
        grid=(GRID,),
        in_specs=[
            pl.BlockSpec((BLK, D), lambda i: (i, 0)),
            pl.BlockSpec((D, D), lambda i: (0, 0)),
            pl.BlockSpec((D, D), lambda i: (0, 0)),
        ],
        out_specs=(
            pl.BlockSpec((BLK, D), lambda i: (i, 0)),
            pl.BlockSpec((BLK, D), lambda i: (i, 0)),
        ),
    )(x, wl, wr)


def _k2_body(p_ref, cnt_ref, xr_ref, bl_ref, gw_ref, asrc_ref, adst_ref,
             hg_ref, as_ref, ad_ref):
    p = p_ref[0] + p_ref[1]
    cnt = cnt_ref[0, :, 0] + cnt_ref[1, :, 0]
    mean = p / jnp.maximum(cnt, 1.0)[:, None]
    h = jnp.maximum(mean + bl_ref[...][None, :] + xr_ref[...], 0.0)
    dn = (((1,), (1,)), ((), ()))
    hg = lax.dot_general(h, gw_ref[...], dn, preferred_element_type=jnp.float32)
    hg_ref[...] = hg
    dv = (((1,), (0,)), ((), ()))
    as_ref[...] = lax.dot_general(hg, asrc_ref[...], dv, preferred_element_type=jnp.float32)[:, None]
    ad_ref[...] = lax.dot_general(hg, adst_ref[...], dv, preferred_element_type=jnp.float32)[:, None]


def _k2(acc_p, cnt_p, xr, bl, gw, asrc, adst):
    return pl.pallas_call(
        _k2_body,
        out_shape=(
            jax.ShapeDtypeStruct((N, D), jnp.float32),
            jax.ShapeDtypeStruct((N, 1), jnp.float32),
            jax.ShapeDtypeStruct((N, 1), jnp.float32),
        ),
        grid=(GRID,),
        in_specs=[
            pl.BlockSpec((2, BLK, D), lambda i: (0, i, 0)),
            pl.BlockSpec((2, BLK, 1), lambda i: (0, i, 0)),
            pl.BlockSpec((BLK, D), lambda i: (i, 0)),
            pl.BlockSpec((D,), lambda i: (0,)),
            pl.BlockSpec((D, D), lambda i: (0, 0)),
            pl.BlockSpec((D,), lambda i: (0,)),
            pl.BlockSpec((D,), lambda i: (0,)),
        ],
        out_specs=(
            pl.BlockSpec((BLK, D), lambda i: (i, 0)),
            pl.BlockSpec((BLK, 1), lambda i: (i, 0)),
            pl.BlockSpec((BLK, 1), lambda i: (i, 0)),
        ),
    )(acc_p, cnt_p, xr, bl, gw, asrc, adst)


def _k2b_body(as_ref, ad_ref, wl_ref, ma_ref):
    a_s = as_ref[...]
    a_d = ad_ref[...]
    max_a = jnp.max(a_s)
    t = a_s + a_d
    el = jnp.where(t > 0, t, 0.2 * t)
    t2 = max_a + a_d
    m = jnp.where(t2 > 0, t2, 0.2 * t2)
    wl_ref[...] = jnp.exp(el - m)
    ma_ref[...] = jnp.full((1, 16), max_a, jnp.float32)


def _k2b(a_s, a_d):
    return pl.pallas_call(
        _k2b_body,
        out_shape=(
            jax.ShapeDtypeStruct((N, 1), jnp.float32),
            jax.ShapeDtypeStruct((1, 16), jnp.float32),
        ),
    )(a_s, a_d)


def _k3_body(s_ref, den_ref, wl_ref, hg_ref, bias_ref, o_ref):
    wl = wl_ref[...]
    num = s_ref[0] + s_ref[1] + wl * hg_ref[...]
    den = den_ref[0, :, 0] + den_ref[1, :, 0] + wl[:, 0] + 1e-16
    out = num / den[:, None] + bias_ref[...][None, :]
    mx = jnp.max(out, axis=1, keepdims=True)
    o_ref[...] = out - mx - jnp.log(jnp.sum(jnp.exp(out - mx), axis=1, keepdims=True))


def _k3(gacc_p, den_p, w_loop, hg, bias):
    return pl.pallas_call(
        _k3_body,
        out_shape=jax.ShapeDtypeStruct((N, D), jnp.float32),
        grid=(GRID,),
        in_specs=[
            pl.BlockSpec((2, BLK, D), lambda i: (0, i, 0)),
            pl.BlockSpec((2, BLK, 1), lambda i: (0, i, 0)),
            pl.BlockSpec((BLK, 1), lambda i: (i, 0)),
            pl.BlockSpec((BLK, D), lambda i: (i, 0)),
            pl.BlockSpec((D,), lambda i: (0,)),
        ],
        out_specs=pl.BlockSpec((BLK, D), lambda i: (i, 0)),
    )(gacc_p, den_p, w_loop, hg, bias)


def kernel(x, edge_index, sage_w_l, sage_b_l, sage_w_r, gat_w, gat_att_src, gat_att_dst, gat_bias):
    src = edge_index[0].astype(jnp.int32)
    dst = edge_index[1].astype(jnp.int32)
    pad = jnp.full((E_ROWS * CH - E,), DUMMY, jnp.int32)
    srcc = jnp.concatenate([src, pad]).reshape(E_ROWS, CH)
    dstc = jnp.concatenate([dst, pad]).reshape(E_ROWS, CH)

    xl, xr = _k1(x, sage_w_l, sage_w_r)
    xl_pad = jnp.pad(xl, ((0, N1 - N), (0, 0)))
    acc_p, cnt_p = _sc_sage(xl_pad, srcc, dstc)

    hg, a_s, a_d = _k2(acc_p, cnt_p.reshape(2, N1, 1), xr, sage_b_l,
                       gat_w, gat_att_src, gat_att_dst)
    w_loop, maxa = _k2b(a_s, a_d)

    hg_pad = jnp.pad(hg, ((0, N1 - N), (0, 0)))
    as_pad = jnp.pad(a_s[:, 0], (0, N1 - N))
    ad_pad = jnp.pad(a_d[:, 0], (0, N1 - N))
    gacc_p, den_p = _sc_gat(hg_pad, srcc, dstc, as_pad, ad_pad, maxa[0])

    return _k3(gacc_p, den_p.reshape(2, N1, 1), w_loop, hg, gat_bias)
